# Initial kernel scaffold; baseline (speedup 1.0000x reference)
import jax
import jax.numpy as jnp
from jax.experimental import pallas as pl


def kernel(x, p00, p01, p02, p03, p04, p05, p06, p07, p08, p09, p10, p11, p12, p13, p14, p15, p16, p17, p18, p19, p20, p21, p22, p23, p24, p25, p26, p27, p28, p29, p30, p31, p32, p33, p34, p35, p36, p37, p38, p39, p40, p41, p42, p43, p44, p45, p46, p47, p48, p49):
    raise NotImplementedError("write your pallas kernel here")


# speedup vs baseline: 4.4611x; 4.4611x over previous
import functools

import jax
import jax.numpy as jnp
from jax import lax
from jax.experimental import pallas as pl
from jax.experimental.pallas import tpu as pltpu

_BN_EPS = 1e-5
_CDT = jnp.bfloat16


def _conv_tap_kernel(x_ref, w_ref, b_ref, *out_refs,
                     taps_per_q, h_tile, wo, act, stats, kw, cin):
    if stats:
        o_ref, sum_ref, ssq_ref = out_refs
    else:
        (o_ref,) = out_refs
    i0 = pl.program_id(0) * h_tile
    s_acc = None
    for q, taps in enumerate(taps_per_q):
        acc = None
        for (p, oi, oj, t) in taps:
            if kw > 0:
                xs = jnp.concatenate(
                    [x_ref[p, pl.ds(i0 + oi, h_tile), pl.ds(oj + dj, wo), :]
                     for dj in range(kw)], axis=-1).reshape(h_tile * wo, kw * cin)
            else:
                xs = x_ref[p, pl.ds(i0 + oi, h_tile),
                           pl.ds(oj, wo), :].reshape(h_tile * wo, cin)
            d = jnp.dot(xs.astype(_CDT), w_ref[t],
                        preferred_element_type=jnp.float32)
            acc = d if acc is None else acc + d
        acc = acc + b_ref[...]
        if act == "relu":
            acc = jnp.maximum(acc, 0.0)
        elif act == "tanh":
            acc = jnp.tanh(acc)
        if stats:
            s = jnp.sum(acc, axis=0, keepdims=True)
            ss = jnp.sum(acc * acc, axis=0, keepdims=True)
            s_acc = (s, ss) if s_acc is None else (s_acc[0] + s, s_acc[1] + ss)
        o_ref[q] = acc.reshape(h_tile, wo, -1).astype(o_ref.dtype)
    if stats:
        sum_ref[0] = s_acc[0]
        ssq_ref[0] = s_acc[1]


def _affine_kernel(*refs, act, with_res):
    if with_res:
        x_ref, s_ref, t_ref, r_ref, o_ref = refs
    else:
        x_ref, s_ref, t_ref, o_ref = refs
    y = x_ref[...].astype(jnp.float32) * s_ref[...] + t_ref[...]
    if act == "relu":
        y = jnp.maximum(y, 0.0)
    if with_res:
        y = y + r_ref[...].astype(jnp.float32)
    o_ref[...] = y.astype(o_ref.dtype)


def _conv_pallas(x4, w3, bias, *, taps_per_q, ho, wo, cout, act, stats,
                 kw=0, out_dtype=jnp.float32):
    P, Hp, Wp, Cin = x4.shape
    Q = len(taps_per_q)
    h_tile = 16 if ho % 16 == 0 else ho
    n = ho // h_tile
    out_shapes = [jax.ShapeDtypeStruct((Q, ho, wo, cout), out_dtype)]
    out_specs = [pl.BlockSpec((Q, h_tile, wo, cout), lambda i: (0, i, 0, 0))]
    if stats:
        out_shapes += [jax.ShapeDtypeStruct((n, 1, cout), jnp.float32)] * 2
        out_specs += [pl.BlockSpec((1, 1, cout), lambda i: (i, 0, 0))] * 2
    res = pl.pallas_call(
        functools.partial(_conv_tap_kernel, taps_per_q=taps_per_q,
                          h_tile=h_tile, wo=wo, act=act, stats=stats,
                          kw=kw, cin=Cin),
        out_shape=tuple(out_shapes),
        grid_spec=pltpu.PrefetchScalarGridSpec(
            num_scalar_prefetch=0,
            grid=(n,),
            in_specs=[
                pl.BlockSpec((P, Hp, Wp, Cin), lambda i: (0, 0, 0, 0)),
                pl.BlockSpec(w3.shape, lambda i: (0, 0, 0)),
                pl.BlockSpec((1, cout), lambda i: (0, 0)),
            ],
            out_specs=tuple(out_specs)),
        compiler_params=pltpu.CompilerParams(
            dimension_semantics=("parallel",)),
    )(x4, w3, bias.reshape(1, -1).astype(jnp.float32))
    return res if stats else res[0]


def _affine(y2, scale, shift, act, residual=None, out_dtype=jnp.float32):
    rows, c = y2.shape
    if rows >= 8192:
        rt = rows // 8
    elif rows >= 2048:
        rt = rows // 4
    else:
        rt = rows
    in_specs = [
        pl.BlockSpec((rt, c), lambda i: (i, 0)),
        pl.BlockSpec((1, c), lambda i: (0, 0)),
        pl.BlockSpec((1, c), lambda i: (0, 0)),
    ]
    args = [y2, scale.reshape(1, -1), shift.reshape(1, -1)]
    if residual is not None:
        in_specs.append(pl.BlockSpec((rt, c), lambda i: (i, 0)))
        args.append(residual)
    return pl.pallas_call(
        functools.partial(_affine_kernel, act=act,
                          with_res=residual is not None),
        out_shape=jax.ShapeDtypeStruct((rows, c), out_dtype),
        grid_spec=pltpu.PrefetchScalarGridSpec(
            num_scalar_prefetch=0,
            grid=(rows // rt,),
            in_specs=in_specs,
            out_specs=pl.BlockSpec((rt, c), lambda i: (i, 0))),
        compiler_params=pltpu.CompilerParams(
            dimension_semantics=("parallel",)),
    )(*args)


def _bn_apply(y, sums, ssqs, gamma, beta, act, residual=None):
    Q, H, W, C = y.shape
    rows = Q * H * W
    s = jnp.sum(sums, axis=0).reshape(-1)
    ss = jnp.sum(ssqs, axis=0).reshape(-1)
    mean = s / rows
    var = jnp.maximum(ss / rows - mean * mean, 0.0)
    scale = gamma * lax.rsqrt(var + _BN_EPS)
    shift = beta - mean * scale
    r2 = None if residual is None else residual.reshape(rows, C)
    z = _affine(y.reshape(rows, C), scale, shift, act, r2)
    return z.reshape(Q, H, W, C)


def _w_taps3(w):
    return jnp.transpose(w, (2, 3, 1, 0)).reshape(
        9, w.shape[1], w.shape[0]).astype(_CDT)


def _w_taps7(w):
    return jnp.transpose(w, (2, 3, 1, 0)).reshape(
        7, 7 * w.shape[1], w.shape[0]).astype(_CDT)


def _w_tapsT(w):
    return jnp.transpose(w[:, :, ::-1, ::-1], (2, 3, 0, 1)).reshape(
        9, w.shape[0], w.shape[1]).astype(_CDT)


def _conv7_layer(x, p, act_out, use_bn=True):
    H, W, _ = x.shape
    cout = p["w"].shape[0]
    xp = jnp.pad(x, ((3, 3), (3, 3), (0, 0)), mode="reflect")[None]
    taps = [[(0, di, 0, di) for di in range(7)]]
    if use_bn:
        y, s, ss = _conv_pallas(xp, _w_taps7(p["w"]), p["b"], taps_per_q=taps,
                                ho=H, wo=W, cout=cout, act="none", stats=True,
                                kw=7)
        return _bn_apply(y, s, ss, p["gamma"], p["beta"], act_out)[0]
    y = _conv_pallas(xp, _w_taps7(p["w"]), p["b"], taps_per_q=taps,
                     ho=H, wo=W, cout=cout, act=act_out, stats=False,
                     kw=7, out_dtype=jnp.float32)
    return y[0]


def _conv3_s1(x, p, act_out, residual=None):
    H, W, _ = x.shape
    cout = p["w"].shape[0]
    xp = jnp.pad(x, ((1, 1), (1, 1), (0, 0)), mode="reflect")[None]
    taps = [[(0, di, dj, di * 3 + dj) for di in range(3) for dj in range(3)]]
    y, s, ss = _conv_pallas(xp, _w_taps3(p["w"]), p["b"], taps_per_q=taps,
                            ho=H, wo=W, cout=cout, act="none", stats=True)
    r = None if residual is None else residual[None]
    return _bn_apply(y, s, ss, p["gamma"], p["beta"], act_out, residual=r)[0]


def _conv3_s2(x, p, residual=None):
    H, W, Cin = x.shape
    cout = p["w"].shape[0]
    xp = jnp.pad(x, ((1, 1), (1, 1), (0, 0)))
    hp, wp = H + 2, W + 2
    ph = xp.reshape(hp // 2, 2, wp // 2, 2, Cin).transpose(
        1, 3, 0, 2, 4).reshape(4, hp // 2, wp // 2, Cin)
    taps = [[((di % 2) * 2 + (dj % 2), di // 2, dj // 2, di * 3 + dj)
             for di in range(3) for dj in range(3)]]
    y, s, ss = _conv_pallas(ph, _w_taps3(p["w"]), p["b"], taps_per_q=taps,
                            ho=H // 2, wo=W // 2, cout=cout, act="none",
                            stats=True)
    r = None if residual is None else residual[None]
    return _bn_apply(y, s, ss, p["gamma"], p["beta"], "relu", residual=r)[0]


def _convT(x, p):
    H, W, _ = x.shape
    cout = p["w"].shape[1]
    xp = jnp.pad(x, ((0, 1), (0, 1), (0, 0)))[None]
    rowt = {0: [(1, 0)], 1: [(0, 0), (2, 1)]}
    taps = [[(0, oi, oj, dr * 3 + dc)
             for (dr, oi) in rowt[a] for (dc, oj) in rowt[b]]
            for a in (0, 1) for b in (0, 1)]
    y, s, ss = _conv_pallas(xp, _w_tapsT(p["w"]), p["b"], taps_per_q=taps,
                            ho=H, wo=W, cout=cout, act="none", stats=True)
    z = _bn_apply(y, s, ss, p["gamma"], p["beta"], "relu")
    return z.reshape(2, 2, H, W, cout).transpose(2, 0, 3, 1, 4).reshape(
        2 * H, 2 * W, cout)


def _avgpool(x):
    H, W, C = x.shape
    ho, wo = (H - 1) // 2 + 1, (W - 1) // 2 + 1
    xp = jnp.pad(x, ((1, 1), (1, 1), (0, 0)))
    ones = jnp.pad(jnp.ones((H, W, 1), x.dtype), ((1, 1), (1, 1), (0, 0)))
    acc = jnp.zeros((ho, wo, C), x.dtype)
    cnt = jnp.zeros((ho, wo, 1), x.dtype)
    for di in range(3):
        for dj in range(3):
            acc = acc + lax.slice(xp, (di, dj, 0),
                                  (di + (ho - 1) * 2 + 1,
                                   dj + (wo - 1) * 2 + 1, C), (2, 2, 1))
            cnt = cnt + lax.slice(ones, (di, dj, 0),
                                  (di + (ho - 1) * 2 + 1,
                                   dj + (wo - 1) * 2 + 1, 1), (2, 2, 1))
    return acc / cnt


def _forward(x_nchw, params):
    g = params["global"]
    loc = params["locals"][0]
    x0 = jnp.transpose(x_nchw, (0, 2, 3, 1))[0]
    x1 = _avgpool(x0)
    x0b = x0
    x1b = x1

    y = _conv7_layer(x1b, g["head"], "relu")
    y = _conv3_s2(y, g["down"][0])
    y = _conv3_s2(y, g["down"][1])
    for bp in g["blocks"]:
        z = _conv3_s1(y, bp["conv1"], "relu")
        y = _conv3_s1(z, bp["conv2"], "none", residual=y)
    y = _convT(y, g["up"][0])
    out = _convT(y, g["up"][1])

    d = _conv7_layer(x0b, loc["down1"], "relu")
    y = _conv3_s2(d, loc["down2"], residual=out)
    for bp in loc["blocks"]:
        z = _conv3_s1(y, bp["conv1"], "relu")
        y = _conv3_s1(z, bp["conv2"], "none", residual=y)
    y = _convT(y, loc["up"])
    y = _conv7_layer(y, loc["tail"], "tanh", use_bn=False)
    return jnp.transpose(y, (2, 0, 1))[None]


def kernel(x, p00, p01, p02, p03, p04, p05, p06, p07, p08, p09,
           p10, p11, p12, p13, p14, p15, p16, p17, p18, p19,
           p20, p21, p22, p23, p24, p25, p26, p27, p28, p29,
           p30, p31, p32, p33, p34, p35, p36, p37, p38, p39,
           p40, p41, p42, p43, p44, p45, p46, p47, p48, p49):
    leaves = [p00, p01, p02, p03, p04, p05, p06, p07, p08, p09,
              p10, p11, p12, p13, p14, p15, p16, p17, p18, p19,
              p20, p21, p22, p23, p24, p25, p26, p27, p28, p29,
              p30, p31, p32, p33, p34, p35, p36, p37, p38, p39,
              p40, p41, p42, p43, p44, p45, p46, p47, p48, p49]

    def _cb():
        return {"w": 0, "b": 0, "gamma": 0, "beta": 0}

    g = {"head": _cb(),
         "down": [_cb(), _cb()],
         "blocks": [{"conv1": _cb(), "conv2": _cb()}],
         "up": [_cb(), _cb()]}
    locs = [{"down1": _cb(), "down2": _cb(),
             "blocks": [{"conv1": _cb(), "conv2": _cb()}],
             "up": _cb(),
             "tail": {"w": 0, "b": 0}}]
    template = {"global": g, "locals": locs}
    treedef = jax.tree_util.tree_structure(template)
    params = jax.tree_util.tree_unflatten(treedef, leaves)
    return _forward(x, params)
